# baseline (speedup 1.0000x reference)
import functools

import jax
import jax.numpy as jnp
from jax import lax
from jax.experimental import pallas as pl
from jax.experimental.pallas import tpu as pltpu
from jax.experimental.pallas import tpu_sc as plsc

_N = 10000
_D = 128
_DH = 64
_G = 64
_C = 10
_E = 320000
_NC = 2
_NS = 16
_LANE = 128
_NCH = 160
_NBUF = 4
_EPAD = _NS * _NCH * _LANE
_PADE = _EPAD - _E
_R = 10000
_SPAN = 624
_TAIL = _R - _NS * _SPAN
_RD = 10240
_SPD = _RD // _NS
_XROWS = _N + 8


@functools.lru_cache(maxsize=None)
def _build_deg_kernel():
    mesh = plsc.VectorSubcoreMesh(core_axis_name="c", subcore_axis_name="s")
    return functools.partial(
        pl.kernel,
        mesh=mesh,
        out_type=jax.ShapeDtypeStruct((_NC, _RD), jnp.float32),
        scratch_types=[
            pltpu.VMEM((_NCH, _LANE), jnp.int32),
            pltpu.VMEM((_LANE,), jnp.float32),
            pltpu.VMEM((_SPD,), jnp.float32),
            pltpu.VMEM_SHARED((_RD,), jnp.float32),
        ],
    )(_deg_body)


def _deg_body(dst_hbm, out_hbm, dstv, onesv, zv, dega):
    cid = lax.axis_index("c")
    sid = lax.axis_index("s")
    pltpu.sync_copy(dst_hbm.at[sid], dstv)
    for j in range(_LANE // 16):
        onesv[pl.ds(j * 16, 16)] = jnp.ones((16,), jnp.float32)

    def _zero(i, c):
        zv[pl.ds(i * 16, 16)] = jnp.zeros((16,), jnp.float32)
        return c

    lax.fori_loop(0, _SPD // 16, _zero, 0)
    pltpu.sync_copy(zv, dega.at[pl.ds(sid * _SPD, _SPD)])
    plsc.subcore_barrier()

    def _chunk(j, c):
        pltpu.sync_copy(onesv, dega.at[dstv.at[cid * (_NCH // 2) + j]],
                        add=True)
        return c

    lax.fori_loop(0, _NCH // 2, _chunk, 0)
    plsc.subcore_barrier()
    pltpu.sync_copy(dega.at[pl.ds(sid * _SPD, _SPD)], zv)
    pltpu.sync_copy(zv, out_hbm.at[cid, pl.ds(sid * _SPD, _SPD)])


@functools.lru_cache(maxsize=None)
def _build_msg_kernel():
    mesh = plsc.VectorSubcoreMesh(core_axis_name="c", subcore_axis_name="s")
    return functools.partial(
        pl.kernel,
        mesh=mesh,
        out_type=jax.ShapeDtypeStruct((_NC, _R, _DH), jnp.float32),
        compiler_params=pltpu.CompilerParams(use_tc_tiling_on_sc=False),
        scratch_types=[
            pltpu.VMEM((_NCH, _LANE), jnp.int32),
            pltpu.VMEM((_NCH, _LANE), jnp.int32),
            pltpu.VMEM((_LANE, _DH), jnp.float32),
            pltpu.VMEM((_LANE, _DH), jnp.float32),
            pltpu.VMEM((_LANE, _DH), jnp.float32),
            pltpu.VMEM((_LANE, _DH), jnp.float32),
            pltpu.VMEM_SHARED((_R, _DH), jnp.float32),
            pltpu.SemaphoreType.DMA,
            pltpu.SemaphoreType.DMA,
            pltpu.SemaphoreType.DMA,
            pltpu.SemaphoreType.DMA,
        ],
    )(_msg_body)


def _msg_body(src_hbm, dst_hbm, xw_hbm, out_hbm, srcv, dstv, r0, r1, r2, r3,
              acc, s0, s1, s2, s3):
    bufs = [r0, r1, r2, r3]
    sems = [s0, s1, s2, s3]
    cid = lax.axis_index("c")
    sid = lax.axis_index("s")
    last = sid == _NS - 1
    pltpu.sync_copy(src_hbm.at[cid, sid], srcv)
    pltpu.sync_copy(dst_hbm.at[sid], dstv)

    def _zero(i, c):
        for j in range(_DH // 16):
            r0[i, pl.ds(j * 16, 16)] = jnp.zeros((16,), jnp.float32)
        return c

    lax.fori_loop(0, _LANE, _zero, 0)
    nz = _SPAN // _LANE
    rz = _SPAN - nz * _LANE
    for t in range(nz):
        pltpu.sync_copy(r0, acc.at[pl.ds(sid * _SPAN + t * _LANE, _LANE)])
    pltpu.sync_copy(r0.at[pl.ds(0, rz)],
                    acc.at[pl.ds(sid * _SPAN + nz * _LANE, rz)])

    @pl.when(last)
    def _():
        pltpu.sync_copy(r0.at[pl.ds(0, _TAIL)],
                        acc.at[pl.ds(_NS * _SPAN, _TAIL)])

    plsc.subcore_barrier()

    def _group(g, c):
        base = g * _NBUF
        gh = [pltpu.async_copy(xw_hbm.at[srcv.at[base + b]], bufs[b], sems[b])
              for b in range(_NBUF)]
        for b in range(_NBUF):
            gh[b].wait()
            pltpu.sync_copy(bufs[b], acc.at[dstv.at[base + b]], add=True)
        return c

    lax.fori_loop(0, _NCH // _NBUF, _group, 0)
    plsc.subcore_barrier()
    for t in range(nz):
        sl = pl.ds(sid * _SPAN + t * _LANE, _LANE)
        pltpu.sync_copy(acc.at[sl], r0)
        pltpu.sync_copy(r0, out_hbm.at[cid, sl])
    sl = pl.ds(sid * _SPAN + nz * _LANE, rz)
    pltpu.sync_copy(acc.at[sl], r0.at[pl.ds(0, rz)])
    pltpu.sync_copy(r0.at[pl.ds(0, rz)], out_hbm.at[cid, sl])

    @pl.when(last)
    def _():
        sl2 = pl.ds(_NS * _SPAN, _TAIL)
        pltpu.sync_copy(acc.at[sl2], r0.at[pl.ds(0, _TAIL)])
        pltpu.sync_copy(r0.at[pl.ds(0, _TAIL)], out_hbm.at[cid, sl2])


def _deg_dinv(degt_ref, rows):
    deg = degt_ref[:, 0:1] + degt_ref[:, 1:2] + 1.0
    iota = lax.broadcasted_iota(jnp.int32, (rows, 1), 0)
    deg = deg - jnp.where(iota < _PADE, 1.0, 0.0)
    return lax.rsqrt(deg)


def _split_halves(o_ref, v):
    o_ref[0] = v[:, :_DH]
    o_ref[1] = v[:, _DH:]


def _scale_body(degt_ref, x_ref, w_ref, o_ref):
    dinv = _deg_dinv(degt_ref, _XROWS)
    xw = jnp.dot(x_ref[...], w_ref[...], preferred_element_type=jnp.float32)
    _split_halves(o_ref, xw * dinv)


def _layer_body(a_ref, xws_ref, degt_ref, b_ref, w_ref, o_ref):
    dinv = _deg_dinv(degt_ref, _XROWS)
    agg = jnp.concatenate([a_ref[0], a_ref[1]], axis=1)
    aggp = jnp.concatenate(
        [agg, jnp.zeros((_XROWS - _N, _D), jnp.float32)], axis=0)
    xws = jnp.concatenate([xws_ref[0], xws_ref[1]], axis=1)
    h = (aggp + xws) * dinv + b_ref[...]
    iota = lax.broadcasted_iota(jnp.int32, (_XROWS, 1), 0)
    h = jnp.maximum(h, 0.0) * jnp.where(iota < _N, 1.0, 0.0)
    o = jnp.dot(h, w_ref[...], preferred_element_type=jnp.float32) * dinv
    _split_halves(o_ref, o)


def _head_body(a_ref, xws_ref, degt_ref, b_ref, batch_ref, wfc_ref,
               bfc_ref, o_ref):
    dinv = _deg_dinv(degt_ref, _XROWS)[:_N, :]
    agg = jnp.concatenate([a_ref[0], a_ref[1]], axis=1)
    xws = jnp.concatenate([xws_ref[0, :_N], xws_ref[1, :_N]], axis=1)
    h = (agg + xws) * dinv + b_ref[...]
    h = jnp.maximum(h, 0.0)
    gids = lax.broadcasted_iota(jnp.int32, (_G, _N), 0)
    onehot = (batch_ref[...] == gids).astype(jnp.float32)
    sums = jnp.dot(onehot, h, preferred_element_type=jnp.float32)
    cnts = jnp.sum(onehot, axis=1, keepdims=True)
    g = sums / jnp.maximum(cnts, 1.0)
    logits = jnp.dot(g, wfc_ref[...], preferred_element_type=jnp.float32)
    logits = logits + bfc_ref[...]
    m = jnp.max(logits, axis=1, keepdims=True)
    s = logits - m
    o_ref[...] = s - jnp.log(jnp.sum(jnp.exp(s), axis=1, keepdims=True))


def _tc_call(body, out_shape, *args):
    return pl.pallas_call(
        body, out_shape=jax.ShapeDtypeStruct(out_shape, jnp.float32))(*args)


@jax.jit
def kernel(x, edge_index, batch, W1, b1, W2, b2, Wfc, bfc):
    src = edge_index[0].astype(jnp.int32)
    dst = edge_index[1].astype(jnp.int32)
    padi = jnp.arange(_PADE, dtype=jnp.int32)
    srcp = jnp.concatenate([src, _N + padi % (_XROWS - _N)])
    dstp = jnp.concatenate([dst, padi])
    srcp = srcp.reshape(_NS, _NCH, _LANE)
    srcp = jnp.stack([srcp, srcp + _XROWS])
    dstp = dstp.reshape(_NS, _NCH, _LANE)
    xp = jnp.concatenate([x, jnp.zeros((_XROWS - _N, _D), x.dtype)])

    deg_parts = _build_deg_kernel()(dstp)
    degt = jnp.transpose(deg_parts[:, :_N])
    degt = jnp.concatenate([degt, jnp.zeros((_XROWS - _N, 2), jnp.float32)])

    msg = _build_msg_kernel()
    xw1s = _tc_call(_scale_body, (_NC, _XROWS, _DH), degt, xp, W1)
    acc1 = msg(srcp, dstp, xw1s.reshape(_NC * _XROWS, _DH))
    xw2s = _tc_call(_layer_body, (_NC, _XROWS, _DH),
                    acc1, xw1s, degt, b1.reshape(1, _D), W2)
    acc2 = msg(srcp, dstp, xw2s.reshape(_NC * _XROWS, _DH))
    out = _tc_call(_head_body, (_G, _C),
                   acc2, xw2s, degt,
                   b2.reshape(1, _D), batch.astype(jnp.int32).reshape(1, _N),
                   Wfc, bfc.reshape(1, _C))
    return out


# speedup vs baseline: 1.0078x; 1.0078x over previous
import functools

import jax
import jax.numpy as jnp
from jax import lax
from jax.experimental import pallas as pl
from jax.experimental.pallas import tpu as pltpu
from jax.experimental.pallas import tpu_sc as plsc

_N = 10000
_D = 128
_G = 64
_C = 10
_E = 320000
_NC = 2
_NS = 16
_NW = _NC * _NS
_LANE = 128
_NCH = 80
_EPAD = _NW * _NCH * _LANE
_PADE = _EPAD - _E
_R = 10000
_SPAN = 624
_TAIL = _R - _NS * _SPAN
_RD = 10240
_SPD = _RD // _NS
_XROWS = _N + 8


@functools.lru_cache(maxsize=None)
def _build_deg_kernel():
    mesh = plsc.VectorSubcoreMesh(core_axis_name="c", subcore_axis_name="s")
    return functools.partial(
        pl.kernel,
        mesh=mesh,
        out_type=jax.ShapeDtypeStruct((_NC, _RD), jnp.float32),
        scratch_types=[
            pltpu.VMEM((_NCH, _LANE), jnp.int32),
            pltpu.VMEM((_LANE,), jnp.float32),
            pltpu.VMEM((_SPD,), jnp.float32),
            pltpu.VMEM_SHARED((_RD,), jnp.float32),
        ],
    )(_deg_body)


def _deg_body(dst_hbm, out_hbm, dstv, onesv, zv, dega):
    cid = lax.axis_index("c")
    sid = lax.axis_index("s")
    wid = sid * _NC + cid
    pltpu.sync_copy(dst_hbm.at[wid], dstv)
    for j in range(_LANE // 16):
        onesv[pl.ds(j * 16, 16)] = jnp.ones((16,), jnp.float32)

    def _zero(i, c):
        zv[pl.ds(i * 16, 16)] = jnp.zeros((16,), jnp.float32)
        return c

    lax.fori_loop(0, _SPD // 16, _zero, 0)
    pltpu.sync_copy(zv, dega.at[pl.ds(sid * _SPD, _SPD)])
    plsc.subcore_barrier()

    def _chunk(j, c):
        pltpu.sync_copy(onesv, dega.at[dstv.at[j]], add=True)
        return c

    lax.fori_loop(0, _NCH, _chunk, 0)
    plsc.subcore_barrier()
    pltpu.sync_copy(dega.at[pl.ds(sid * _SPD, _SPD)], zv)
    pltpu.sync_copy(zv, out_hbm.at[cid, pl.ds(sid * _SPD, _SPD)])


@functools.lru_cache(maxsize=None)
def _build_msg_kernel():
    mesh = plsc.VectorSubcoreMesh(core_axis_name="c", subcore_axis_name="s")
    return functools.partial(
        pl.kernel,
        mesh=mesh,
        out_type=jax.ShapeDtypeStruct((_NC, _R, _D), jnp.float32),
        scratch_types=[
            pltpu.VMEM((_NCH, _LANE), jnp.int32),
            pltpu.VMEM((_NCH, _LANE), jnp.int32),
            pltpu.VMEM((_LANE, _D), jnp.float32),
            pltpu.VMEM((_LANE, _D), jnp.float32),
            pltpu.VMEM_SHARED((_R, _D), jnp.float32),
            pltpu.SemaphoreType.DMA,
            pltpu.SemaphoreType.DMA,
        ],
    )(_msg_body)


def _msg_body(src_hbm, dst_hbm, xw_hbm, out_hbm, srcv, dstv, r0, r1,
              acc, s0, s1):
    cid = lax.axis_index("c")
    sid = lax.axis_index("s")
    wid = sid * _NC + cid
    last = sid == _NS - 1
    pltpu.sync_copy(src_hbm.at[wid], srcv)
    pltpu.sync_copy(dst_hbm.at[wid], dstv)

    def _zero(i, c):
        for j in range(_D // 16):
            r0[i, pl.ds(j * 16, 16)] = jnp.zeros((16,), jnp.float32)
        return c

    lax.fori_loop(0, _LANE, _zero, 0)
    nz = _SPAN // _LANE
    rz = _SPAN - nz * _LANE
    for t in range(nz):
        pltpu.sync_copy(r0, acc.at[pl.ds(sid * _SPAN + t * _LANE, _LANE)])
    pltpu.sync_copy(r0.at[pl.ds(0, rz)],
                    acc.at[pl.ds(sid * _SPAN + nz * _LANE, rz)])

    @pl.when(last)
    def _():
        pltpu.sync_copy(r0.at[pl.ds(0, _TAIL)],
                        acc.at[pl.ds(_NS * _SPAN, _TAIL)])

    plsc.subcore_barrier()

    def _chunk(j, c):
        pltpu.async_copy(xw_hbm.at[srcv.at[j]], r0, s0).wait()
        pltpu.sync_copy(r0, acc.at[dstv.at[j]], add=True)
        return c

    lax.fori_loop(0, _NCH, _chunk, 0)
    plsc.subcore_barrier()
    for t in range(nz):
        sl = pl.ds(sid * _SPAN + t * _LANE, _LANE)
        pltpu.sync_copy(acc.at[sl], r0)
        pltpu.sync_copy(r0, out_hbm.at[cid, sl])
    sl = pl.ds(sid * _SPAN + nz * _LANE, rz)
    pltpu.sync_copy(acc.at[sl], r0.at[pl.ds(0, rz)])
    pltpu.sync_copy(r0.at[pl.ds(0, rz)], out_hbm.at[cid, sl])

    @pl.when(last)
    def _():
        sl2 = pl.ds(_NS * _SPAN, _TAIL)
        pltpu.sync_copy(acc.at[sl2], r0.at[pl.ds(0, _TAIL)])
        pltpu.sync_copy(r0.at[pl.ds(0, _TAIL)], out_hbm.at[cid, sl2])


def _deg_dinv(degt_ref, rows):
    deg = degt_ref[:, 0:1] + degt_ref[:, 1:2] + 1.0
    iota = lax.broadcasted_iota(jnp.int32, (rows, 1), 0)
    deg = deg - jnp.where(iota < _PADE, 1.0, 0.0)
    return lax.rsqrt(deg)


def _scale_body(degt_ref, x_ref, w_ref, o_ref):
    dinv = _deg_dinv(degt_ref, _XROWS)
    xw = jnp.dot(x_ref[...], w_ref[...], preferred_element_type=jnp.float32)
    o_ref[...] = xw * dinv


def _layer_body(a0_ref, a1_ref, xws_ref, degt_ref, b_ref, w_ref, o_ref):
    dinv = _deg_dinv(degt_ref, _XROWS)
    agg = a0_ref[...] + a1_ref[...]
    aggp = jnp.concatenate(
        [agg, jnp.zeros((_XROWS - _N, _D), jnp.float32)], axis=0)
    h = (aggp + xws_ref[...]) * dinv + b_ref[...]
    iota = lax.broadcasted_iota(jnp.int32, (_XROWS, 1), 0)
    h = jnp.maximum(h, 0.0) * jnp.where(iota < _N, 1.0, 0.0)
    o_ref[...] = jnp.dot(h, w_ref[...], preferred_element_type=jnp.float32) * dinv


def _head_body(a0_ref, a1_ref, xws_ref, degt_ref, b_ref, batch_ref, wfc_ref,
               bfc_ref, o_ref):
    dinv = _deg_dinv(degt_ref, _XROWS)[:_N, :]
    h = (a0_ref[...] + a1_ref[...] + xws_ref[:_N, :]) * dinv + b_ref[...]
    h = jnp.maximum(h, 0.0)
    gids = lax.broadcasted_iota(jnp.int32, (_G, _N), 0)
    onehot = (batch_ref[...] == gids).astype(jnp.float32)
    sums = jnp.dot(onehot, h, preferred_element_type=jnp.float32)
    cnts = jnp.sum(onehot, axis=1, keepdims=True)
    g = sums / jnp.maximum(cnts, 1.0)
    logits = jnp.dot(g, wfc_ref[...], preferred_element_type=jnp.float32)
    logits = logits + bfc_ref[...]
    m = jnp.max(logits, axis=1, keepdims=True)
    s = logits - m
    o_ref[...] = s - jnp.log(jnp.sum(jnp.exp(s), axis=1, keepdims=True))


def _tc_call(body, out_shape, *args):
    return pl.pallas_call(
        body, out_shape=jax.ShapeDtypeStruct(out_shape, jnp.float32))(*args)


@jax.jit
def kernel(x, edge_index, batch, W1, b1, W2, b2, Wfc, bfc):
    src = edge_index[0].astype(jnp.int32)
    dst = edge_index[1].astype(jnp.int32)
    padi = jnp.arange(_PADE, dtype=jnp.int32)
    srcp = jnp.concatenate([src, _N + padi % (_XROWS - _N)])
    dstp = jnp.concatenate([dst, padi])
    srcp = jnp.transpose(srcp.reshape(_NCH, _NW, _LANE), (1, 0, 2))
    dstp = jnp.transpose(dstp.reshape(_NCH, _NW, _LANE), (1, 0, 2))
    xp = jnp.concatenate([x, jnp.zeros((_XROWS - _N, _D), x.dtype)])

    deg_parts = _build_deg_kernel()(dstp)
    degt = jnp.transpose(deg_parts[:, :_N])
    degt = jnp.concatenate([degt, jnp.zeros((_XROWS - _N, 2), jnp.float32)])

    msg = _build_msg_kernel()
    xw1s = _tc_call(_scale_body, (_XROWS, _D), degt, xp, W1)
    acc1 = msg(srcp, dstp, xw1s)
    xw2s = _tc_call(_layer_body, (_XROWS, _D),
                    acc1[0], acc1[1], xw1s, degt, b1.reshape(1, _D), W2)
    acc2 = msg(srcp, dstp, xw2s)
    out = _tc_call(_head_body, (_G, _C),
                   acc2[0], acc2[1], xw2s, degt,
                   b2.reshape(1, _D), batch.astype(jnp.int32).reshape(1, _N),
                   Wfc, bfc.reshape(1, _C))
    return out


# speedup vs baseline: 1.0172x; 1.0093x over previous
import functools

import jax
import jax.numpy as jnp
from jax import lax
from jax.experimental import pallas as pl
from jax.experimental.pallas import tpu as pltpu
from jax.experimental.pallas import tpu_sc as plsc

_N = 10000
_D = 128
_G = 64
_C = 10
_E = 320000
_NC = 2
_NS = 16
_NW = _NC * _NS
_LANE = 128
_NCH = 80
_EPAD = _NW * _NCH * _LANE
_PADE = _EPAD - _E
_R = 10000
_SPAN = 624
_TAIL = _R - _NS * _SPAN
_RD = 10240
_SPD = _RD // _NS
_XROWS = _N + 8


@functools.lru_cache(maxsize=None)
def _build_deg_kernel():
    mesh = plsc.VectorSubcoreMesh(core_axis_name="c", subcore_axis_name="s")
    return functools.partial(
        pl.kernel,
        mesh=mesh,
        out_type=jax.ShapeDtypeStruct((_NC, _RD), jnp.float32),
        scratch_types=[
            pltpu.VMEM((_NCH, _LANE), jnp.int32),
            pltpu.VMEM((_LANE,), jnp.float32),
            pltpu.VMEM((_SPD,), jnp.float32),
            pltpu.VMEM_SHARED((_RD,), jnp.float32),
        ],
    )(_deg_body)


def _deg_body(dst_hbm, out_hbm, dstv, onesv, zv, dega):
    cid = lax.axis_index("c")
    sid = lax.axis_index("s")
    wid = sid * _NC + cid
    pltpu.sync_copy(dst_hbm.at[wid], dstv)
    for j in range(_LANE // 16):
        onesv[pl.ds(j * 16, 16)] = jnp.ones((16,), jnp.float32)

    def _zero(i, c):
        zv[pl.ds(i * 16, 16)] = jnp.zeros((16,), jnp.float32)
        return c

    lax.fori_loop(0, _SPD // 16, _zero, 0)
    pltpu.sync_copy(zv, dega.at[pl.ds(sid * _SPD, _SPD)])
    plsc.subcore_barrier()

    def _chunk(j, c):
        pltpu.sync_copy(onesv, dega.at[dstv.at[j]], add=True)
        return c

    lax.fori_loop(0, _NCH, _chunk, 0)
    plsc.subcore_barrier()
    pltpu.sync_copy(dega.at[pl.ds(sid * _SPD, _SPD)], zv)
    pltpu.sync_copy(zv, out_hbm.at[cid, pl.ds(sid * _SPD, _SPD)])


@functools.lru_cache(maxsize=None)
def _build_msg_kernel():
    mesh = plsc.VectorSubcoreMesh(core_axis_name="c", subcore_axis_name="s")
    return functools.partial(
        pl.kernel,
        mesh=mesh,
        out_type=jax.ShapeDtypeStruct((_NC, _R, _D), jnp.float32),
        scratch_types=[
            pltpu.VMEM((_NCH, _LANE), jnp.int32),
            pltpu.VMEM((_NCH, _LANE), jnp.int32),
            pltpu.VMEM((_LANE, _D), jnp.float32),
            pltpu.VMEM((_LANE, _D), jnp.float32),
            pltpu.VMEM_SHARED((_R, _D), jnp.float32),
            pltpu.SemaphoreType.DMA,
            pltpu.SemaphoreType.DMA,
        ],
    )(_msg_body)


def _msg_body(src_hbm, dst_hbm, xw_hbm, out_hbm, srcv, dstv, r0, r1,
              acc, s0, s1):
    cid = lax.axis_index("c")
    sid = lax.axis_index("s")
    wid = sid * _NC + cid
    last = sid == _NS - 1
    pltpu.sync_copy(src_hbm.at[wid], srcv)
    pltpu.sync_copy(dst_hbm.at[wid], dstv)

    def _zero(i, c):
        for j in range(_D // 16):
            r0[i, pl.ds(j * 16, 16)] = jnp.zeros((16,), jnp.float32)
        return c

    lax.fori_loop(0, _LANE, _zero, 0)
    nz = _SPAN // _LANE
    rz = _SPAN - nz * _LANE
    for t in range(nz):
        pltpu.sync_copy(r0, acc.at[pl.ds(sid * _SPAN + t * _LANE, _LANE)])
    pltpu.sync_copy(r0.at[pl.ds(0, rz)],
                    acc.at[pl.ds(sid * _SPAN + nz * _LANE, rz)])

    @pl.when(last)
    def _():
        pltpu.sync_copy(r0.at[pl.ds(0, _TAIL)],
                        acc.at[pl.ds(_NS * _SPAN, _TAIL)])

    plsc.subcore_barrier()

    def _chunk(j, c):
        pltpu.async_copy(xw_hbm.at[srcv.at[j]], r0, s0).wait()
        pltpu.sync_copy(r0, acc.at[dstv.at[j]], add=True)
        return c

    lax.fori_loop(0, _NCH, _chunk, 0)
    plsc.subcore_barrier()
    for t in range(nz):
        sl = pl.ds(sid * _SPAN + t * _LANE, _LANE)
        pltpu.sync_copy(acc.at[sl], r0)
        pltpu.sync_copy(r0, out_hbm.at[cid, sl])
    sl = pl.ds(sid * _SPAN + nz * _LANE, rz)
    pltpu.sync_copy(acc.at[sl], r0.at[pl.ds(0, rz)])
    pltpu.sync_copy(r0.at[pl.ds(0, rz)], out_hbm.at[cid, sl])

    @pl.when(last)
    def _():
        sl2 = pl.ds(_NS * _SPAN, _TAIL)
        pltpu.sync_copy(acc.at[sl2], r0.at[pl.ds(0, _TAIL)])
        pltpu.sync_copy(r0.at[pl.ds(0, _TAIL)], out_hbm.at[cid, sl2])


def _deg_dinv(degt_ref, rows):
    deg = degt_ref[:, 0:1] + degt_ref[:, 1:2] + 1.0
    iota = lax.broadcasted_iota(jnp.int32, (rows, 1), 0)
    deg = deg - jnp.where(iota < _PADE, 1.0, 0.0)
    return lax.rsqrt(deg)


def _scale_body(degt_ref, x_ref, w_ref, o_ref):
    dinv = _deg_dinv(degt_ref, _XROWS)
    xw = jnp.dot(x_ref[...], w_ref[...], preferred_element_type=jnp.float32)
    o_ref[...] = xw * dinv


def _layer_body(a0_ref, a1_ref, xws_ref, degt_ref, b_ref, w_ref, o_ref):
    dinv = _deg_dinv(degt_ref, _XROWS)
    agg = a0_ref[...] + a1_ref[...]
    aggp = jnp.concatenate(
        [agg, jnp.zeros((_XROWS - _N, _D), jnp.float32)], axis=0)
    h = (aggp + xws_ref[...]) * dinv + b_ref[...]
    iota = lax.broadcasted_iota(jnp.int32, (_XROWS, 1), 0)
    h = jnp.maximum(h, 0.0) * jnp.where(iota < _N, 1.0, 0.0)
    o_ref[...] = jnp.dot(h, w_ref[...], preferred_element_type=jnp.float32) * dinv


def _head_body(a0_ref, a1_ref, xws_ref, degt_ref, b_ref, batch_ref, wfc_ref,
               bfc_ref, o_ref):
    dinv = _deg_dinv(degt_ref, _XROWS)[:_N, :]
    h = (a0_ref[...] + a1_ref[...] + xws_ref[:_N, :]) * dinv + b_ref[...]
    h = jnp.maximum(h, 0.0)
    gids = lax.broadcasted_iota(jnp.int32, (_G, _N), 0)
    onehot = (batch_ref[...] == gids).astype(jnp.float32)
    sums = jnp.dot(onehot, h, preferred_element_type=jnp.float32)
    cnts = jnp.sum(onehot, axis=1, keepdims=True)
    g = sums / jnp.maximum(cnts, 1.0)
    logits = jnp.dot(g, wfc_ref[...], preferred_element_type=jnp.float32)
    logits = logits + bfc_ref[...]
    m = jnp.max(logits, axis=1, keepdims=True)
    s = logits - m
    o_ref[...] = s - jnp.log(jnp.sum(jnp.exp(s), axis=1, keepdims=True))


def _tc_call(body, out_shape, *args):
    return pl.pallas_call(
        body, out_shape=jax.ShapeDtypeStruct(out_shape, jnp.float32))(*args)


@jax.jit
def kernel(x, edge_index, batch, W1, b1, W2, b2, Wfc, bfc):
    src = edge_index[0].astype(jnp.int32)
    dst = edge_index[1].astype(jnp.int32)
    padi = jnp.arange(_PADE, dtype=jnp.int32)
    srcp = jnp.concatenate([src, _N + padi % (_XROWS - _N)])
    dstp = jnp.concatenate([dst, padi])
    srcp = srcp.reshape(_NW, _NCH, _LANE)
    dstp = dstp.reshape(_NW, _NCH, _LANE)
    xp = jnp.concatenate([x, jnp.zeros((_XROWS - _N, _D), x.dtype)])

    deg_parts = _build_deg_kernel()(dstp)
    degt = jnp.transpose(deg_parts[:, :_N])
    degt = jnp.concatenate([degt, jnp.zeros((_XROWS - _N, 2), jnp.float32)])

    msg = _build_msg_kernel()
    xw1s = _tc_call(_scale_body, (_XROWS, _D), degt, xp, W1)
    acc1 = msg(srcp, dstp, xw1s)
    xw2s = _tc_call(_layer_body, (_XROWS, _D),
                    acc1[0], acc1[1], xw1s, degt, b1.reshape(1, _D), W2)
    acc2 = msg(srcp, dstp, xw2s)
    out = _tc_call(_head_body, (_G, _C),
                   acc2[0], acc2[1], xw2s, degt,
                   b2.reshape(1, _D), batch.astype(jnp.int32).reshape(1, _N),
                   Wfc, bfc.reshape(1, _C))
    return out
